# Initial kernel scaffold; baseline (speedup 1.0000x reference)
import jax
import jax.numpy as jnp
from jax.experimental import pallas as pl


def kernel(x, edge_index, W1, a_src1, a_dst1, b1, W2, a_src2, a_dst2, b2):
    raise NotImplementedError("write your pallas kernel here")


# speedup vs baseline: 30.0802x; 30.0802x over previous
import functools

import jax
import jax.numpy as jnp
from jax import lax
from jax.experimental import pallas as pl
from jax.experimental.pallas import tpu as pltpu
from jax.experimental.pallas import tpu_sc as plsc

N_NODES = 10000
D_IN = 128
D_HID = 128
N_CLASSES = 3
N_EDGES = 320000

N_PAD = 10240
E_REAL = N_EDGES + N_NODES
NW = 32
EPW = 10368
E_PAD = NW * EPW
G = 128
NGRP = EPW // G
RPT = N_PAD // 16


def _tc_pre_body(x_ref, w_ref, a2_ref, h_ref, ap_ref):
    h = jnp.dot(x_ref[...], w_ref[...], preferred_element_type=jnp.float32)
    h_ref[...] = h
    ap_ref[...] = jnp.dot(h, a2_ref[...], preferred_element_type=jnp.float32)


def _tc_pre(x_pad, W1, a2):
    br = 1024
    return pl.pallas_call(
        _tc_pre_body,
        grid=(N_PAD // br,),
        in_specs=[
            pl.BlockSpec((br, D_IN), lambda i: (i, 0)),
            pl.BlockSpec((D_IN, D_HID), lambda i: (0, 0)),
            pl.BlockSpec((D_HID, 2), lambda i: (0, 0)),
        ],
        out_specs=[
            pl.BlockSpec((br, D_HID), lambda i: (i, 0)),
            pl.BlockSpec((br, 2), lambda i: (i, 0)),
        ],
        out_shape=[
            jax.ShapeDtypeStruct((N_PAD, D_HID), jnp.float32),
            jax.ShapeDtypeStruct((N_PAD, 2), jnp.float32),
        ],
    )(x_pad, W1, a2)


def _sc1_body(src_hbm, dst_hbm, h_hbm, ap_hbm,
              msg_hbm, den_hbm,
              ap_v, rows_v, sidx_v, didx_v, w_v, den_v,
              acc_sh, sem):
    c = lax.axis_index("c")
    s = lax.axis_index("s")
    wid = c * 16 + s
    base = wid * EPW

    pltpu.sync_copy(ap_hbm, ap_v)

    zf = jnp.zeros((16,), jnp.float32)

    def zden(i, _):
        den_v[pl.ds(i * 16, 16)] = zf
        return 0
    lax.fori_loop(0, N_PAD // 16, zden, 0)

    def zrow(i, _):
        for k in range(8):
            rows_v[i, pl.ds(k * 16, 16)] = zf
        return 0
    lax.fori_loop(0, G, zrow, 0)

    for k in range(RPT // G):
        pltpu.sync_copy(rows_v, acc_sh.at[pl.ds(s * RPT + k * G, G)])
    plsc.subcore_barrier()

    lane = jnp.arange(16, dtype=jnp.int32)

    def group(g, _):
        eb = base + g * G
        pltpu.sync_copy(src_hbm.at[pl.ds(eb, G)], sidx_v)
        pltpu.sync_copy(dst_hbm.at[pl.ds(eb, G)], didx_v)
        pltpu.async_copy(h_hbm.at[sidx_v], rows_v, sem).wait()
        for j in range(G // 16):
            si = sidx_v[pl.ds(j * 16, 16)]
            di = didx_v[pl.ds(j * 16, 16)]
            a = (plsc.load_gather(ap_v, [si * 2])
                 + plsc.load_gather(ap_v, [di * 2 + 1]))
            a = jnp.maximum(a, 0.2 * a)
            eid = eb + j * 16 + lane
            w = jnp.where(eid < E_REAL, jnp.exp(a), 0.0)
            w_v[pl.ds(j * 16, 16)] = w
            plsc.addupdate_scatter(den_v, [di], w)

        def scale(j, _):
            wspl = plsc.load_gather(w_v, [jnp.full((16,), j, jnp.int32)])
            for k in range(8):
                rows_v[j, pl.ds(k * 16, 16)] = rows_v[j, pl.ds(k * 16, 16)] * wspl
            return 0
        lax.fori_loop(0, G, scale, 0)

        pltpu.sync_copy(rows_v, acc_sh.at[didx_v], add=True)
        return 0

    lax.fori_loop(0, NGRP, group, 0)
    plsc.subcore_barrier()

    pltpu.sync_copy(acc_sh.at[pl.ds(s * RPT, RPT)],
                    msg_hbm.at[c].at[pl.ds(s * RPT, RPT)])
    pltpu.sync_copy(den_v, den_hbm.at[wid])


@functools.partial(
    pl.kernel,
    out_type=(
        jax.ShapeDtypeStruct((2, N_PAD, D_HID), jnp.float32),
        jax.ShapeDtypeStruct((NW, N_PAD), jnp.float32),
    ),
    mesh=plsc.VectorSubcoreMesh(core_axis_name="c", subcore_axis_name="s"),
    compiler_params=pltpu.CompilerParams(needs_layout_passes=False),
    scratch_types=(
        pltpu.VMEM((N_PAD * 2,), jnp.float32),
        pltpu.VMEM((G, D_HID), jnp.float32),
        pltpu.VMEM((G,), jnp.int32),
        pltpu.VMEM((G,), jnp.int32),
        pltpu.VMEM((G,), jnp.float32),
        pltpu.VMEM((N_PAD,), jnp.float32),
        pltpu.VMEM_SHARED((N_PAD, D_HID), jnp.float32),
        pltpu.SemaphoreType.DMA,
    ),
)
def _sc_layer1(src_hbm, dst_hbm, h_hbm, ap_hbm, msg_hbm, den_hbm,
               ap_v, rows_v, sidx_v, didx_v, w_v, den_v,
               acc_sh, sem):
    _sc1_body(src_hbm, dst_hbm, h_hbm, ap_hbm, msg_hbm, den_hbm,
              ap_v, rows_v, sidx_v, didx_v, w_v, den_v,
              acc_sh, sem)


def _tc_mid_body(msg_ref, den_ref, b1_ref, w2_ref, a22_ref, hp2_ref):
    p = msg_ref[0] + msg_ref[1]
    d = jnp.sum(den_ref[...], axis=0)
    o = p / (d[:, None] + 1e-16) + b1_ref[...]
    o = jnp.maximum(o, 0.0)
    h2 = jnp.dot(o, w2_ref[...], preferred_element_type=jnp.float32)
    a22 = a22_ref[...]
    as2 = jnp.sum(h2 * a22[0:1, :], axis=1)
    ad2 = jnp.sum(h2 * a22[1:2, :], axis=1)
    hp2_ref[...] = jnp.concatenate([h2, as2[:, None], ad2[:, None]], axis=1)


def _tc_mid(msg1, den1, b1_2d, W2, a22):
    br = 1024
    return pl.pallas_call(
        _tc_mid_body,
        grid=(N_PAD // br,),
        in_specs=[
            pl.BlockSpec((2, br, D_HID), lambda i: (0, i, 0)),
            pl.BlockSpec((NW, br), lambda i: (0, i)),
            pl.BlockSpec((1, D_HID), lambda i: (0, 0)),
            pl.BlockSpec((D_HID, N_CLASSES), lambda i: (0, 0)),
            pl.BlockSpec((2, N_CLASSES), lambda i: (0, 0)),
        ],
        out_specs=pl.BlockSpec((br, 5), lambda i: (i, 0)),
        out_shape=jax.ShapeDtypeStruct((N_PAD, 5), jnp.float32),
    )(msg1, den1, b1_2d, W2, a22)


def _sc2_body(src_hbm, dst_hbm, hp2_hbm, msg_hbm, den_hbm,
              hp2_v, src_v, dst_v, acc_v, den_v):
    c = lax.axis_index("c")
    s = lax.axis_index("s")
    wid = c * 16 + s
    base = wid * EPW

    pltpu.sync_copy(hp2_hbm, hp2_v)
    pltpu.sync_copy(src_hbm.at[pl.ds(base, EPW)], src_v)
    pltpu.sync_copy(dst_hbm.at[pl.ds(base, EPW)], dst_v)

    zf = jnp.zeros((16,), jnp.float32)

    def zden(i, _):
        den_v[pl.ds(i * 16, 16)] = zf
        return 0
    lax.fori_loop(0, N_PAD // 16, zden, 0)

    def zacc(i, _):
        acc_v[pl.ds(i * 16, 16)] = zf
        return 0
    lax.fori_loop(0, N_PAD * 4 // 16, zacc, 0)

    lane = jnp.arange(16, dtype=jnp.int32)

    def step(i, _):
        e0 = i * 16
        si = src_v[pl.ds(e0, 16)]
        di = dst_v[pl.ds(e0, 16)]
        si5 = si * 5
        a = (plsc.load_gather(hp2_v, [si5 + 3])
             + plsc.load_gather(hp2_v, [di * 5 + 4]))
        a = jnp.maximum(a, 0.2 * a)
        eid = base + e0 + lane
        w = jnp.where(eid < E_REAL, jnp.exp(a), 0.0)
        plsc.addupdate_scatter(den_v, [di], w)
        di4 = di * 4
        for col in range(N_CLASSES):
            hv = plsc.load_gather(hp2_v, [si5 + col])
            plsc.addupdate_scatter(acc_v, [di4 + col], w * hv)
        return 0

    lax.fori_loop(0, EPW // 16, step, 0)

    pltpu.sync_copy(acc_v, msg_hbm.at[wid])
    pltpu.sync_copy(den_v, den_hbm.at[wid])


@functools.partial(
    pl.kernel,
    out_type=(
        jax.ShapeDtypeStruct((NW, N_PAD * 4), jnp.float32),
        jax.ShapeDtypeStruct((NW, N_PAD), jnp.float32),
    ),
    mesh=plsc.VectorSubcoreMesh(core_axis_name="c", subcore_axis_name="s"),
    compiler_params=pltpu.CompilerParams(needs_layout_passes=False),
    scratch_types=(
        pltpu.VMEM((N_PAD * 5,), jnp.float32),
        pltpu.VMEM((EPW,), jnp.int32),
        pltpu.VMEM((EPW,), jnp.int32),
        pltpu.VMEM((N_PAD * 4,), jnp.float32),
        pltpu.VMEM((N_PAD,), jnp.float32),
    ),
)
def _sc_layer2(src_hbm, dst_hbm, hp2_hbm, msg_hbm, den_hbm,
               hp2_v, src_v, dst_v, acc_v, den_v):
    _sc2_body(src_hbm, dst_hbm, hp2_hbm, msg_hbm, den_hbm,
              hp2_v, src_v, dst_v, acc_v, den_v)


def _tc_final_body(msg_ref, den_ref, b2_ref, out_ref):
    sm = jnp.sum(msg_ref[...], axis=0)
    d = jnp.sum(den_ref[...], axis=0)
    out_ref[...] = sm / (d[:, None] + 1e-16) + b2_ref[...]


def _tc_final(msg2, den2, b2p):
    br = 1024
    return pl.pallas_call(
        _tc_final_body,
        grid=(N_PAD // br,),
        in_specs=[
            pl.BlockSpec((NW, br, 4), lambda i: (0, i, 0)),
            pl.BlockSpec((NW, br), lambda i: (0, i)),
            pl.BlockSpec((1, 4), lambda i: (0, 0)),
        ],
        out_specs=pl.BlockSpec((br, 4), lambda i: (i, 0)),
        out_shape=jax.ShapeDtypeStruct((N_PAD, 4), jnp.float32),
    )(msg2, den2, b2p)


def kernel(x, edge_index, W1, a_src1, a_dst1, b1, W2, a_src2, a_dst2, b2):
    loop = jnp.arange(N_NODES, dtype=edge_index.dtype)
    src = jnp.concatenate([edge_index[0], loop])
    dst = jnp.concatenate([edge_index[1], loop])
    src_p = jnp.zeros((E_PAD,), jnp.int32).at[:E_REAL].set(src.astype(jnp.int32))
    dst_p = jnp.zeros((E_PAD,), jnp.int32).at[:E_REAL].set(dst.astype(jnp.int32))

    x_pad = jnp.zeros((N_PAD, D_IN), jnp.float32).at[:N_NODES].set(x)
    a2 = jnp.stack([a_src1, a_dst1], axis=1)
    a22 = jnp.stack([a_src2, a_dst2], axis=0)
    b1_2d = b1[None, :]
    b2p = jnp.concatenate([b2, jnp.zeros((1,), jnp.float32)])[None, :]

    h, ap = _tc_pre(x_pad, W1, a2)
    msg1, den1 = _sc_layer1(src_p, dst_p, h, ap.reshape(N_PAD * 2))
    hp2 = _tc_mid(msg1, den1, b1_2d, W2, a22)
    msg2, den2 = _sc_layer2(src_p, dst_p, hp2.reshape(N_PAD * 5))
    out = _tc_final(msg2.reshape(NW, N_PAD, 4), den2, b2p)
    return out[:N_NODES, :N_CLASSES]


# speedup vs baseline: 32.7183x; 1.0877x over previous
import functools

import jax
import jax.numpy as jnp
from jax import lax
from jax.experimental import pallas as pl
from jax.experimental.pallas import tpu as pltpu
from jax.experimental.pallas import tpu_sc as plsc

N_NODES = 10000
D_IN = 128
D_HID = 128
N_CLASSES = 3
N_EDGES = 320000

N_PAD = 10240
E_REAL = N_EDGES + N_NODES
NW = 32
EPW = 10368
E_PAD = NW * EPW
G = 64
NGRP = EPW // G
RPT = N_PAD // 16


def _tc_pre_body(x_ref, w_ref, a2_ref, h_ref, ap_ref):
    h = jnp.dot(x_ref[...], w_ref[...], preferred_element_type=jnp.float32)
    h_ref[...] = h
    ap_ref[...] = jnp.dot(h, a2_ref[...], preferred_element_type=jnp.float32)


def _tc_pre(x_pad, W1, a2):
    br = 1024
    return pl.pallas_call(
        _tc_pre_body,
        grid=(N_PAD // br,),
        in_specs=[
            pl.BlockSpec((br, D_IN), lambda i: (i, 0)),
            pl.BlockSpec((D_IN, D_HID), lambda i: (0, 0)),
            pl.BlockSpec((D_HID, 2), lambda i: (0, 0)),
        ],
        out_specs=[
            pl.BlockSpec((br, D_HID), lambda i: (i, 0)),
            pl.BlockSpec((br, 2), lambda i: (i, 0)),
        ],
        out_shape=[
            jax.ShapeDtypeStruct((N_PAD, D_HID), jnp.float32),
            jax.ShapeDtypeStruct((N_PAD, 2), jnp.float32),
        ],
    )(x_pad, W1, a2)


def _sc1_body(sd_hbm, h_hbm, ap_hbm,
              msg_hbm, den_hbm,
              ap_v, rows_v, sd_v, w_v, zidx_v, den_v,
              acc_sh, gs0, gs1, ss0, ss1, is0, is1, is2, is3):
    c = lax.axis_index("c")
    s = lax.axis_index("s")
    wid = c * 16 + s
    gg0 = wid * NGRP
    lane = jnp.arange(16, dtype=jnp.int32)
    gsem = (gs0, gs1)
    ssem = (ss0, ss1)
    isem = (is0, is1, is2, is3)

    pltpu.sync_copy(ap_hbm, ap_v)

    zf = jnp.zeros((16,), jnp.float32)

    def zden(i, _):
        den_v[pl.ds(i * 16, 16)] = zf
        return 0
    lax.fori_loop(0, N_PAD // 16, zden, 0)

    def zrow(i, _):
        for k in range(8):
            rows_v[0, i, pl.ds(k * 16, 16)] = zf
            rows_v[1, i, pl.ds(k * 16, 16)] = zf
        return 0
    lax.fori_loop(0, G, zrow, 0)

    for k in range(RPT // G):
        pltpu.sync_copy(rows_v.at[0], acc_sh.at[pl.ds(s * RPT + k * G, G)])
    plsc.subcore_barrier()

    zi = jnp.full((16,), N_PAD - 1, jnp.int32)
    for j in range(G // 16):
        zidx_v[pl.ds(j * 16, 16)] = zi

    def idx_start(gg, b):
        pltpu.async_copy(sd_hbm.at[gg], sd_v.at[b], isem[b])

    def idx_wait(b):
        pltpu.make_async_copy(sd_hbm.at[0], sd_v.at[b], isem[b]).wait()

    def gather_start(b, p):
        pltpu.async_copy(h_hbm.at[sd_v.at[b].at[0]], rows_v.at[p], gsem[p])

    def gather_wait(p):
        pltpu.make_async_copy(h_hbm.at[sd_v.at[0].at[0]], rows_v.at[p],
                              gsem[p]).wait()

    def scatter_start(b, p):
        pltpu.async_copy(rows_v.at[p], acc_sh.at[sd_v.at[b].at[1]], ssem[p],
                         add=True)

    def scatter_wait(p):
        pltpu.make_async_copy(rows_v.at[p], acc_sh.at[sd_v.at[0].at[1]],
                              ssem[p]).wait()

    def w_compute(b, p, gg):
        for j in range(G // 16):
            si = sd_v[b, 0, pl.ds(j * 16, 16)]
            di = sd_v[b, 1, pl.ds(j * 16, 16)]
            a = (plsc.load_gather(ap_v, [si * 2])
                 + plsc.load_gather(ap_v, [di * 2 + 1]))
            a = jnp.maximum(a, 0.2 * a)
            eid = gg * G + j * 16 + lane
            w = jnp.where(eid < E_REAL, jnp.exp(a), 0.0)
            w_v[pl.ds(p * G + j * 16, 16)] = w
            plsc.addupdate_scatter(den_v, [di], w)

    def scale(p):
        def body(j, _):
            wspl = plsc.load_gather(w_v, [jnp.full((16,), p * G + j, jnp.int32)])
            for k in range(8):
                rows_v[p, j, pl.ds(k * 16, 16)] = (
                    rows_v[p, j, pl.ds(k * 16, 16)] * wspl)
            return 0
        lax.fori_loop(0, G, body, 0)

    idx_start(gg0 + 0, 0)
    idx_start(gg0 + 1, 1)
    idx_start(gg0 + 2, 2)
    pltpu.async_copy(rows_v.at[1], acc_sh.at[zidx_v], ss1, add=True)
    idx_wait(0)
    gather_start(0, 0)

    gather_wait(0)
    w_compute(0, 0, gg0)
    scale(0)
    scatter_start(0, 0)
    scatter_wait(1)
    idx_wait(1)
    gather_start(1, 1)
    idx_start(gg0 + 3, 3)

    gather_wait(1)
    w_compute(1, 1, gg0 + 1)
    scale(1)
    scatter_start(1, 1)
    scatter_wait(0)
    idx_wait(2)
    gather_start(2, 0)
    idx_start(gg0 + 4, 0)

    def quad(k, _):
        g = 2 + 4 * k
        for u in range(4):
            b = (2 + u) % 4
            p = u % 2
            gather_wait(p)
            w_compute(b, p, gg0 + g + u)
            scale(p)
            scatter_start(b, p)
            scatter_wait(1 - p)
            idx_wait((b + 1) % 4)
            gather_start((b + 1) % 4, 1 - p)
            idx_start(gg0 + g + u + 3, (b + 3) % 4)
        return 0
    lax.fori_loop(0, (NGRP - 2) // 4, quad, 0)

    scatter_wait(1)
    gather_wait(0)
    idx_wait(3)
    idx_wait(0)

    plsc.subcore_barrier()
    pltpu.sync_copy(acc_sh.at[pl.ds(s * RPT, RPT)],
                    msg_hbm.at[c].at[pl.ds(s * RPT, RPT)])
    pltpu.sync_copy(den_v, den_hbm.at[wid])


@functools.partial(
    pl.kernel,
    out_type=(
        jax.ShapeDtypeStruct((2, N_PAD, D_HID), jnp.float32),
        jax.ShapeDtypeStruct((NW, N_PAD), jnp.float32),
    ),
    mesh=plsc.VectorSubcoreMesh(core_axis_name="c", subcore_axis_name="s"),
    compiler_params=pltpu.CompilerParams(needs_layout_passes=False),
    scratch_types=(
        pltpu.VMEM((N_PAD * 2,), jnp.float32),
        pltpu.VMEM((2, G, D_HID), jnp.float32),
        pltpu.VMEM((4, 2, G), jnp.int32),
        pltpu.VMEM((2 * G,), jnp.float32),
        pltpu.VMEM((G,), jnp.int32),
        pltpu.VMEM((N_PAD,), jnp.float32),
        pltpu.VMEM_SHARED((N_PAD, D_HID), jnp.float32),
        pltpu.SemaphoreType.DMA,
        pltpu.SemaphoreType.DMA,
        pltpu.SemaphoreType.DMA,
        pltpu.SemaphoreType.DMA,
        pltpu.SemaphoreType.DMA,
        pltpu.SemaphoreType.DMA,
        pltpu.SemaphoreType.DMA,
        pltpu.SemaphoreType.DMA,
    ),
)
def _sc_layer1(sd_hbm, h_hbm, ap_hbm, msg_hbm, den_hbm,
               ap_v, rows_v, sd_v, w_v, zidx_v, den_v,
               acc_sh, gs0, gs1, ss0, ss1, is0, is1, is2, is3):
    _sc1_body(sd_hbm, h_hbm, ap_hbm, msg_hbm, den_hbm,
              ap_v, rows_v, sd_v, w_v, zidx_v, den_v,
              acc_sh, gs0, gs1, ss0, ss1, is0, is1, is2, is3)


def _tc_mid_body(msg_ref, den_ref, b1_ref, w2_ref, a22_ref, hp2_ref):
    p = msg_ref[0] + msg_ref[1]
    d = jnp.sum(den_ref[...], axis=0)
    o = p / (d[:, None] + 1e-16) + b1_ref[...]
    o = jnp.maximum(o, 0.0)
    h2 = jnp.dot(o, w2_ref[...], preferred_element_type=jnp.float32)
    a22 = a22_ref[...]
    as2 = jnp.sum(h2 * a22[0:1, :], axis=1)
    ad2 = jnp.sum(h2 * a22[1:2, :], axis=1)
    hp2_ref[...] = jnp.concatenate([h2, as2[:, None], ad2[:, None]], axis=1)


def _tc_mid(msg1, den1, b1_2d, W2, a22):
    br = 1024
    return pl.pallas_call(
        _tc_mid_body,
        grid=(N_PAD // br,),
        in_specs=[
            pl.BlockSpec((2, br, D_HID), lambda i: (0, i, 0)),
            pl.BlockSpec((NW, br), lambda i: (0, i)),
            pl.BlockSpec((1, D_HID), lambda i: (0, 0)),
            pl.BlockSpec((D_HID, N_CLASSES), lambda i: (0, 0)),
            pl.BlockSpec((2, N_CLASSES), lambda i: (0, 0)),
        ],
        out_specs=pl.BlockSpec((br, 5), lambda i: (i, 0)),
        out_shape=jax.ShapeDtypeStruct((N_PAD, 5), jnp.float32),
    )(msg1, den1, b1_2d, W2, a22)


def _sc2_body(src_hbm, dst_hbm, hp2_hbm, msg_hbm, den_hbm,
              hp2_v, src_v, dst_v, acc_v, den_v):
    c = lax.axis_index("c")
    s = lax.axis_index("s")
    wid = c * 16 + s
    base = wid * EPW

    pltpu.sync_copy(hp2_hbm, hp2_v)
    pltpu.sync_copy(src_hbm.at[pl.ds(base, EPW)], src_v)
    pltpu.sync_copy(dst_hbm.at[pl.ds(base, EPW)], dst_v)

    zf = jnp.zeros((16,), jnp.float32)

    def zden(i, _):
        den_v[pl.ds(i * 16, 16)] = zf
        return 0
    lax.fori_loop(0, N_PAD // 16, zden, 0)

    def zacc(i, _):
        acc_v[pl.ds(i * 16, 16)] = zf
        return 0
    lax.fori_loop(0, N_PAD * 4 // 16, zacc, 0)

    lane = jnp.arange(16, dtype=jnp.int32)

    def step(i, _):
        e0 = i * 16
        si = src_v[pl.ds(e0, 16)]
        di = dst_v[pl.ds(e0, 16)]
        si5 = si * 5
        a = (plsc.load_gather(hp2_v, [si5 + 3])
             + plsc.load_gather(hp2_v, [di * 5 + 4]))
        a = jnp.maximum(a, 0.2 * a)
        eid = base + e0 + lane
        w = jnp.where(eid < E_REAL, jnp.exp(a), 0.0)
        plsc.addupdate_scatter(den_v, [di], w)
        di4 = di * 4
        for col in range(N_CLASSES):
            hv = plsc.load_gather(hp2_v, [si5 + col])
            plsc.addupdate_scatter(acc_v, [di4 + col], w * hv)
        return 0

    lax.fori_loop(0, EPW // 16, step, 0)

    pltpu.sync_copy(acc_v, msg_hbm.at[wid])
    pltpu.sync_copy(den_v, den_hbm.at[wid])


@functools.partial(
    pl.kernel,
    out_type=(
        jax.ShapeDtypeStruct((NW, N_PAD * 4), jnp.float32),
        jax.ShapeDtypeStruct((NW, N_PAD), jnp.float32),
    ),
    mesh=plsc.VectorSubcoreMesh(core_axis_name="c", subcore_axis_name="s"),
    compiler_params=pltpu.CompilerParams(needs_layout_passes=False),
    scratch_types=(
        pltpu.VMEM((N_PAD * 5,), jnp.float32),
        pltpu.VMEM((EPW,), jnp.int32),
        pltpu.VMEM((EPW,), jnp.int32),
        pltpu.VMEM((N_PAD * 4,), jnp.float32),
        pltpu.VMEM((N_PAD,), jnp.float32),
    ),
)
def _sc_layer2(src_hbm, dst_hbm, hp2_hbm, msg_hbm, den_hbm,
               hp2_v, src_v, dst_v, acc_v, den_v):
    _sc2_body(src_hbm, dst_hbm, hp2_hbm, msg_hbm, den_hbm,
              hp2_v, src_v, dst_v, acc_v, den_v)


def _tc_final_body(msg_ref, den_ref, b2_ref, out_ref):
    sm = jnp.sum(msg_ref[...], axis=0)
    d = jnp.sum(den_ref[...], axis=0)
    out_ref[...] = sm / (d[:, None] + 1e-16) + b2_ref[...]


def _tc_final(msg2, den2, b2p):
    br = 1024
    return pl.pallas_call(
        _tc_final_body,
        grid=(N_PAD // br,),
        in_specs=[
            pl.BlockSpec((NW, br, 4), lambda i: (0, i, 0)),
            pl.BlockSpec((NW, br), lambda i: (0, i)),
            pl.BlockSpec((1, 4), lambda i: (0, 0)),
        ],
        out_specs=pl.BlockSpec((br, 4), lambda i: (i, 0)),
        out_shape=jax.ShapeDtypeStruct((N_PAD, 4), jnp.float32),
    )(msg2, den2, b2p)


def kernel(x, edge_index, W1, a_src1, a_dst1, b1, W2, a_src2, a_dst2, b2):
    loop = jnp.arange(N_NODES, dtype=edge_index.dtype)
    src = jnp.concatenate([edge_index[0], loop])
    dst = jnp.concatenate([edge_index[1], loop])
    src_p = jnp.zeros((E_PAD,), jnp.int32).at[:E_REAL].set(src.astype(jnp.int32))
    dst_p = jnp.zeros((E_PAD,), jnp.int32).at[:E_REAL].set(dst.astype(jnp.int32))
    sd = jnp.stack([src_p.reshape(NW * NGRP, G), dst_p.reshape(NW * NGRP, G)],
                   axis=1)
    sd = jnp.concatenate([sd, jnp.zeros((3, 2, G), jnp.int32)], axis=0)

    x_pad = jnp.zeros((N_PAD, D_IN), jnp.float32).at[:N_NODES].set(x)
    a2 = jnp.stack([a_src1, a_dst1], axis=1)
    a22 = jnp.stack([a_src2, a_dst2], axis=0)
    b1_2d = b1[None, :]
    b2p = jnp.concatenate([b2, jnp.zeros((1,), jnp.float32)])[None, :]

    h, ap = _tc_pre(x_pad, W1, a2)
    msg1, den1 = _sc_layer1(sd, h, ap.reshape(N_PAD * 2))
    hp2 = _tc_mid(msg1, den1, b1_2d, W2, a22)
    msg2, den2 = _sc_layer2(src_p, dst_p, hp2.reshape(N_PAD * 5))
    out = _tc_final(msg2.reshape(NW, N_PAD, 4), den2, b2p)
    return out[:N_NODES, :N_CLASSES]


# speedup vs baseline: 35.0378x; 1.0709x over previous
import functools

import jax
import jax.numpy as jnp
from jax import lax
from jax.experimental import pallas as pl
from jax.experimental.pallas import tpu as pltpu
from jax.experimental.pallas import tpu_sc as plsc

N_NODES = 10000
D_IN = 128
D_HID = 128
N_CLASSES = 3
N_EDGES = 320000

N_PAD = 10240
E_REAL = N_EDGES + N_NODES
NW = 32
EPW = 10368
E_PAD = NW * EPW
G = 64
NGRP = EPW // G
RPT = N_PAD // 16


def _tc_pre_body(x_ref, w_ref, a2_ref, h_ref, ap_ref):
    h = jnp.dot(x_ref[...], w_ref[...], preferred_element_type=jnp.float32)
    h_ref[...] = h
    ap_ref[...] = jnp.dot(h, a2_ref[...], preferred_element_type=jnp.float32)


def _tc_pre(x_pad, W1, a2):
    br = 1024
    return pl.pallas_call(
        _tc_pre_body,
        grid=(N_PAD // br,),
        in_specs=[
            pl.BlockSpec((br, D_IN), lambda i: (i, 0)),
            pl.BlockSpec((D_IN, D_HID), lambda i: (0, 0)),
            pl.BlockSpec((D_HID, 2), lambda i: (0, 0)),
        ],
        out_specs=[
            pl.BlockSpec((br, D_HID), lambda i: (i, 0)),
            pl.BlockSpec((br, 2), lambda i: (i, 0)),
        ],
        out_shape=[
            jax.ShapeDtypeStruct((N_PAD, D_HID), jnp.float32),
            jax.ShapeDtypeStruct((N_PAD, 2), jnp.float32),
        ],
    )(x_pad, W1, a2)


def _sc1_body(sd_hbm, h_hbm, ap_hbm,
              msg_hbm, den_hbm,
              ap_v, rows_v, sd_v, w_v, zidx_v, den_v,
              acc_sh, gs0, gs1, ss0, ss1, is0, is1, is2, is3):
    c = lax.axis_index("c")
    s = lax.axis_index("s")
    wid = c * 16 + s
    gg0 = wid * NGRP
    lane = jnp.arange(16, dtype=jnp.int32)
    gsem = (gs0, gs1)
    ssem = (ss0, ss1)
    isem = (is0, is1, is2, is3)

    pltpu.sync_copy(ap_hbm, ap_v)

    zf = jnp.zeros((16,), jnp.float32)

    def zden(i, _):
        den_v[pl.ds(i * 16, 16)] = zf
        return 0
    lax.fori_loop(0, N_PAD // 16, zden, 0)

    def zrow(i, _):
        for k in range(8):
            rows_v[0, i, pl.ds(k * 16, 16)] = zf
            rows_v[1, i, pl.ds(k * 16, 16)] = zf
        return 0
    lax.fori_loop(0, G, zrow, 0)

    for k in range(RPT // G):
        pltpu.sync_copy(rows_v.at[0], acc_sh.at[pl.ds(s * RPT + k * G, G)])
    plsc.subcore_barrier()

    zi = jnp.full((16,), N_PAD - 1, jnp.int32)
    for j in range(G // 16):
        zidx_v[pl.ds(j * 16, 16)] = zi

    def idx_start(gg, b):
        pltpu.async_copy(sd_hbm.at[gg], sd_v.at[b], isem[b])

    def idx_wait(b):
        pltpu.make_async_copy(sd_hbm.at[0], sd_v.at[b], isem[b]).wait()

    def gather_start(b, p):
        pltpu.async_copy(h_hbm.at[sd_v.at[b].at[0]], rows_v.at[p], gsem[p])

    def gather_wait(p):
        pltpu.make_async_copy(h_hbm.at[sd_v.at[0].at[0]], rows_v.at[p],
                              gsem[p]).wait()

    def scatter_start(b, p):
        pltpu.async_copy(rows_v.at[p], acc_sh.at[sd_v.at[b].at[1]], ssem[p],
                         add=True)

    def scatter_wait(p):
        pltpu.make_async_copy(rows_v.at[p], acc_sh.at[sd_v.at[0].at[1]],
                              ssem[p]).wait()

    def w_compute(b, p, gg):
        for j in range(G // 16):
            si = sd_v[b, 0, pl.ds(j * 16, 16)]
            di = sd_v[b, 1, pl.ds(j * 16, 16)]
            a = (plsc.load_gather(ap_v, [si * 2])
                 + plsc.load_gather(ap_v, [di * 2 + 1]))
            a = jnp.maximum(a, 0.2 * a)
            eid = gg * G + j * 16 + lane
            w = jnp.where(eid < E_REAL, jnp.exp(a), 0.0)
            w_v[pl.ds(p * G + j * 16, 16)] = w
            plsc.addupdate_scatter(den_v, [di], w)

    RU = 8

    def scale(p):
        def body(i, _):
            j0 = i * RU
            spl = [plsc.load_gather(
                w_v, [jnp.full((16,), p * G + j0 + r, jnp.int32)])
                for r in range(RU)]
            for r in range(RU):
                for k in range(8):
                    rows_v[p, j0 + r, pl.ds(k * 16, 16)] = (
                        rows_v[p, j0 + r, pl.ds(k * 16, 16)] * spl[r])
            return 0
        lax.fori_loop(0, G // RU, body, 0)

    idx_start(gg0 + 0, 0)
    idx_start(gg0 + 1, 1)
    idx_start(gg0 + 2, 2)
    pltpu.async_copy(rows_v.at[1], acc_sh.at[zidx_v], ss1, add=True)
    idx_wait(0)
    gather_start(0, 0)

    gather_wait(0)
    w_compute(0, 0, gg0)
    scale(0)
    scatter_start(0, 0)
    scatter_wait(1)
    idx_wait(1)
    gather_start(1, 1)
    idx_start(gg0 + 3, 3)

    gather_wait(1)
    w_compute(1, 1, gg0 + 1)
    scale(1)
    scatter_start(1, 1)
    scatter_wait(0)
    idx_wait(2)
    gather_start(2, 0)
    idx_start(gg0 + 4, 0)

    def quad(k, _):
        g = 2 + 4 * k
        for u in range(4):
            b = (2 + u) % 4
            p = u % 2
            gather_wait(p)
            w_compute(b, p, gg0 + g + u)
            scale(p)
            scatter_start(b, p)
            scatter_wait(1 - p)
            idx_wait((b + 1) % 4)
            gather_start((b + 1) % 4, 1 - p)
            idx_start(gg0 + g + u + 3, (b + 3) % 4)
        return 0
    lax.fori_loop(0, (NGRP - 2) // 4, quad, 0)

    scatter_wait(1)
    gather_wait(0)
    idx_wait(3)
    idx_wait(0)

    plsc.subcore_barrier()
    pltpu.sync_copy(acc_sh.at[pl.ds(s * RPT, RPT)],
                    msg_hbm.at[c].at[pl.ds(s * RPT, RPT)])
    pltpu.sync_copy(den_v, den_hbm.at[wid])


@functools.partial(
    pl.kernel,
    out_type=(
        jax.ShapeDtypeStruct((2, N_PAD, D_HID), jnp.float32),
        jax.ShapeDtypeStruct((NW, N_PAD), jnp.float32),
    ),
    mesh=plsc.VectorSubcoreMesh(core_axis_name="c", subcore_axis_name="s"),
    compiler_params=pltpu.CompilerParams(needs_layout_passes=False),
    scratch_types=(
        pltpu.VMEM((N_PAD * 2,), jnp.float32),
        pltpu.VMEM((2, G, D_HID), jnp.float32),
        pltpu.VMEM((4, 2, G), jnp.int32),
        pltpu.VMEM((2 * G,), jnp.float32),
        pltpu.VMEM((G,), jnp.int32),
        pltpu.VMEM((N_PAD,), jnp.float32),
        pltpu.VMEM_SHARED((N_PAD, D_HID), jnp.float32),
        pltpu.SemaphoreType.DMA,
        pltpu.SemaphoreType.DMA,
        pltpu.SemaphoreType.DMA,
        pltpu.SemaphoreType.DMA,
        pltpu.SemaphoreType.DMA,
        pltpu.SemaphoreType.DMA,
        pltpu.SemaphoreType.DMA,
        pltpu.SemaphoreType.DMA,
    ),
)
def _sc_layer1(sd_hbm, h_hbm, ap_hbm, msg_hbm, den_hbm,
               ap_v, rows_v, sd_v, w_v, zidx_v, den_v,
               acc_sh, gs0, gs1, ss0, ss1, is0, is1, is2, is3):
    _sc1_body(sd_hbm, h_hbm, ap_hbm, msg_hbm, den_hbm,
              ap_v, rows_v, sd_v, w_v, zidx_v, den_v,
              acc_sh, gs0, gs1, ss0, ss1, is0, is1, is2, is3)


def _tc_mid_body(msg_ref, den_ref, b1_ref, w2_ref, a22_ref, hp2_ref):
    p = msg_ref[0] + msg_ref[1]
    d = jnp.sum(den_ref[...], axis=0)
    o = p / (d[:, None] + 1e-16) + b1_ref[...]
    o = jnp.maximum(o, 0.0)
    h2 = jnp.dot(o, w2_ref[...], preferred_element_type=jnp.float32)
    a22 = a22_ref[...]
    as2 = jnp.sum(h2 * a22[0:1, :], axis=1)
    ad2 = jnp.sum(h2 * a22[1:2, :], axis=1)
    hp2_ref[...] = jnp.concatenate([h2, as2[:, None], ad2[:, None]], axis=1)


def _tc_mid(msg1, den1, b1_2d, W2, a22):
    br = 1024
    return pl.pallas_call(
        _tc_mid_body,
        grid=(N_PAD // br,),
        in_specs=[
            pl.BlockSpec((2, br, D_HID), lambda i: (0, i, 0)),
            pl.BlockSpec((NW, br), lambda i: (0, i)),
            pl.BlockSpec((1, D_HID), lambda i: (0, 0)),
            pl.BlockSpec((D_HID, N_CLASSES), lambda i: (0, 0)),
            pl.BlockSpec((2, N_CLASSES), lambda i: (0, 0)),
        ],
        out_specs=pl.BlockSpec((br, 5), lambda i: (i, 0)),
        out_shape=jax.ShapeDtypeStruct((N_PAD, 5), jnp.float32),
    )(msg1, den1, b1_2d, W2, a22)


def _sc2_body(src_hbm, dst_hbm, hp2_hbm, msg_hbm, den_hbm,
              hp2_v, src_v, dst_v, acc_v, den_v):
    c = lax.axis_index("c")
    s = lax.axis_index("s")
    wid = c * 16 + s
    base = wid * EPW

    pltpu.sync_copy(hp2_hbm, hp2_v)
    pltpu.sync_copy(src_hbm.at[pl.ds(base, EPW)], src_v)
    pltpu.sync_copy(dst_hbm.at[pl.ds(base, EPW)], dst_v)

    zf = jnp.zeros((16,), jnp.float32)

    def zden(i, _):
        den_v[pl.ds(i * 16, 16)] = zf
        return 0
    lax.fori_loop(0, N_PAD // 16, zden, 0)

    def zacc(i, _):
        acc_v[pl.ds(i * 16, 16)] = zf
        return 0
    lax.fori_loop(0, N_PAD * 4 // 16, zacc, 0)

    lane = jnp.arange(16, dtype=jnp.int32)

    def step(i, _):
        e0 = i * 16
        si = src_v[pl.ds(e0, 16)]
        di = dst_v[pl.ds(e0, 16)]
        si5 = si * 5
        a = (plsc.load_gather(hp2_v, [si5 + 3])
             + plsc.load_gather(hp2_v, [di * 5 + 4]))
        a = jnp.maximum(a, 0.2 * a)
        eid = base + e0 + lane
        w = jnp.where(eid < E_REAL, jnp.exp(a), 0.0)
        plsc.addupdate_scatter(den_v, [di], w)
        di4 = di * 4
        for col in range(N_CLASSES):
            hv = plsc.load_gather(hp2_v, [si5 + col])
            plsc.addupdate_scatter(acc_v, [di4 + col], w * hv)
        return 0

    lax.fori_loop(0, EPW // 16, step, 0)

    pltpu.sync_copy(acc_v, msg_hbm.at[wid])
    pltpu.sync_copy(den_v, den_hbm.at[wid])


@functools.partial(
    pl.kernel,
    out_type=(
        jax.ShapeDtypeStruct((NW, N_PAD * 4), jnp.float32),
        jax.ShapeDtypeStruct((NW, N_PAD), jnp.float32),
    ),
    mesh=plsc.VectorSubcoreMesh(core_axis_name="c", subcore_axis_name="s"),
    compiler_params=pltpu.CompilerParams(needs_layout_passes=False),
    scratch_types=(
        pltpu.VMEM((N_PAD * 5,), jnp.float32),
        pltpu.VMEM((EPW,), jnp.int32),
        pltpu.VMEM((EPW,), jnp.int32),
        pltpu.VMEM((N_PAD * 4,), jnp.float32),
        pltpu.VMEM((N_PAD,), jnp.float32),
    ),
)
def _sc_layer2(src_hbm, dst_hbm, hp2_hbm, msg_hbm, den_hbm,
               hp2_v, src_v, dst_v, acc_v, den_v):
    _sc2_body(src_hbm, dst_hbm, hp2_hbm, msg_hbm, den_hbm,
              hp2_v, src_v, dst_v, acc_v, den_v)


def _tc_final_body(msg_ref, den_ref, b2_ref, out_ref):
    sm = jnp.sum(msg_ref[...], axis=0)
    d = jnp.sum(den_ref[...], axis=0)
    out_ref[...] = sm / (d[:, None] + 1e-16) + b2_ref[...]


def _tc_final(msg2, den2, b2p):
    br = 1024
    return pl.pallas_call(
        _tc_final_body,
        grid=(N_PAD // br,),
        in_specs=[
            pl.BlockSpec((NW, br, 4), lambda i: (0, i, 0)),
            pl.BlockSpec((NW, br), lambda i: (0, i)),
            pl.BlockSpec((1, 4), lambda i: (0, 0)),
        ],
        out_specs=pl.BlockSpec((br, 4), lambda i: (i, 0)),
        out_shape=jax.ShapeDtypeStruct((N_PAD, 4), jnp.float32),
    )(msg2, den2, b2p)


def kernel(x, edge_index, W1, a_src1, a_dst1, b1, W2, a_src2, a_dst2, b2):
    loop = jnp.arange(N_NODES, dtype=edge_index.dtype)
    src = jnp.concatenate([edge_index[0], loop])
    dst = jnp.concatenate([edge_index[1], loop])
    src_p = jnp.zeros((E_PAD,), jnp.int32).at[:E_REAL].set(src.astype(jnp.int32))
    dst_p = jnp.zeros((E_PAD,), jnp.int32).at[:E_REAL].set(dst.astype(jnp.int32))
    sd = jnp.stack([src_p.reshape(NW * NGRP, G), dst_p.reshape(NW * NGRP, G)],
                   axis=1)
    sd = jnp.concatenate([sd, jnp.zeros((3, 2, G), jnp.int32)], axis=0)

    x_pad = jnp.zeros((N_PAD, D_IN), jnp.float32).at[:N_NODES].set(x)
    a2 = jnp.stack([a_src1, a_dst1], axis=1)
    a22 = jnp.stack([a_src2, a_dst2], axis=0)
    b1_2d = b1[None, :]
    b2p = jnp.concatenate([b2, jnp.zeros((1,), jnp.float32)])[None, :]

    h, ap = _tc_pre(x_pad, W1, a2)
    msg1, den1 = _sc_layer1(sd, h, ap.reshape(N_PAD * 2))
    hp2 = _tc_mid(msg1, den1, b1_2d, W2, a22)
    msg2, den2 = _sc_layer2(src_p, dst_p, hp2.reshape(N_PAD * 5))
    out = _tc_final(msg2.reshape(NW, N_PAD, 4), den2, b2p)
    return out[:N_NODES, :N_CLASSES]
